# baseline (speedup 1.0000x reference)
import functools

import jax
import jax.numpy as jnp
from jax import lax
from jax.experimental import pallas as pl
from jax.experimental.pallas import tpu as pltpu
from jax.experimental.pallas import tpu_sc as plsc

HIDDEN = 768
BATCH = 4
SEQ = 2048
EPS = 1e-12

ROWS = BATCH * SEQ
NC, NS = 2, 16
NW = NC * NS
ROWS_PER_W = ROWS // NW
CHUNK = 64
NCHUNK = ROWS_PER_W // CHUNK


def _gather_body(table_hbm, idx_hbm, out_hbm, idx_v, rows_v, sem):
    wid = lax.axis_index("s") * NC + lax.axis_index("c")
    base = wid * ROWS_PER_W
    pltpu.sync_copy(idx_hbm.at[pl.ds(base, ROWS_PER_W)], idx_v)
    for c in range(NCHUNK):
        buf = rows_v.at[c % 2]
        pltpu.async_copy(
            table_hbm.at[idx_v.at[pl.ds(c * CHUNK, CHUNK)]], buf, sem
        ).wait()
        pltpu.sync_copy(buf, out_hbm.at[pl.ds(base + c * CHUNK, CHUNK)])


@jax.jit
def _sc_gather(table, ids):
    mesh = plsc.VectorSubcoreMesh(core_axis_name="c", subcore_axis_name="s")
    return pl.kernel(
        _gather_body,
        mesh=mesh,
        out_type=jax.ShapeDtypeStruct((ROWS, HIDDEN), jnp.float32),
        scratch_types=[
            pltpu.VMEM((ROWS_PER_W,), jnp.int32),
            pltpu.VMEM((2, CHUNK, HIDDEN), jnp.float32),
            pltpu.SemaphoreType.DMA,
        ],
    )(table, ids)


ROW_BLK = 256
POS_BLKS = SEQ // ROW_BLK


def _ln_body(emb_ref, pos_ref, gamma_ref, beta_ref, out_ref):
    x = emb_ref[...] + pos_ref[...]
    mean = jnp.mean(x, axis=-1, keepdims=True)
    xc = x - mean
    var = jnp.mean(xc * xc, axis=-1, keepdims=True)
    inv = lax.rsqrt(var + EPS)
    out_ref[...] = xc * inv * gamma_ref[...] + beta_ref[...]


@jax.jit
def _tc_layernorm(emb, pos, gamma, beta):
    return pl.pallas_call(
        _ln_body,
        grid=(ROWS // ROW_BLK,),
        in_specs=[
            pl.BlockSpec((ROW_BLK, HIDDEN), lambda i: (i, 0)),
            pl.BlockSpec((ROW_BLK, HIDDEN), lambda i: (i % POS_BLKS, 0)),
            pl.BlockSpec((1, HIDDEN), lambda i: (0, 0)),
            pl.BlockSpec((1, HIDDEN), lambda i: (0, 0)),
        ],
        out_specs=pl.BlockSpec((ROW_BLK, HIDDEN), lambda i: (i, 0)),
        out_shape=jax.ShapeDtypeStruct((ROWS, HIDDEN), jnp.float32),
    )(emb, pos, gamma, beta)


def kernel(input_ids, word_embeddings, position_embeddings, ln_gamma, ln_beta):
    ids = input_ids.reshape(-1).astype(jnp.int32)
    emb = _sc_gather(word_embeddings, ids)
    out = _tc_layernorm(
        emb,
        position_embeddings,
        ln_gamma.reshape(1, HIDDEN),
        ln_beta.reshape(1, HIDDEN),
    )
    return out.reshape(BATCH, SEQ, HIDDEN)
